# Initial kernel scaffold; baseline (speedup 1.0000x reference)
import jax
import jax.numpy as jnp
from jax.experimental import pallas as pl


def kernel(x, edge_index, W1, b1, W2, b2):
    raise NotImplementedError("write your pallas kernel here")


# speedup vs baseline: 18.3275x; 18.3275x over previous
import functools

import jax
import jax.numpy as jnp
from jax import lax
from jax.experimental import pallas as pl
from jax.experimental.pallas import tpu as pltpu
from jax.experimental.pallas import tpu_sc as plsc

NC = 2
NS = 16
LANES = 16
CH = 128


def _sc_mesh():
    return plsc.VectorSubcoreMesh(
        core_axis_name="c", subcore_axis_name="s", num_cores=NC, num_subcores=NS
    )


_SC_PARAMS = pltpu.CompilerParams(use_tc_tiling_on_sc=False)


def _sc_degree(dst_p, zeros1d):
    R = zeros1d.shape[0]
    Rs = R // NS
    e_pad = dst_p.shape[0]
    per_tile = e_pad // (NC * NS)
    n_chunks = per_tile // CH

    @functools.partial(
        pl.kernel,
        out_type=jax.ShapeDtypeStruct((NC * R,), jnp.float32),
        mesh=_sc_mesh(),
        compiler_params=_SC_PARAMS,
        scratch_types=[
            pltpu.VMEM((CH,), jnp.int32),
            pltpu.VMEM((CH,), jnp.float32),
            pltpu.VMEM((Rs,), jnp.float32),
            pltpu.VMEM_SHARED((R,), jnp.float32),
        ],
    )
    def k(dst_hbm, z_hbm, ones_hbm, out_hbm, didx, ones_v, bounce, acc):
        c = lax.axis_index("c")
        s = lax.axis_index("s")
        pltpu.sync_copy(z_hbm.at[pl.ds(s * Rs, Rs)], bounce)
        pltpu.sync_copy(bounce, acc.at[pl.ds(s * Rs, Rs)])
        pltpu.sync_copy(ones_hbm, ones_v)
        plsc.subcore_barrier()
        base = (c * NS + s) * per_tile

        @pl.loop(0, n_chunks)
        def _(i):
            off = base + i * CH
            pltpu.sync_copy(dst_hbm.at[pl.ds(off, CH)], didx)
            pltpu.sync_copy(ones_v, acc.at[didx], add=True)

        plsc.subcore_barrier()
        pltpu.sync_copy(acc.at[pl.ds(s * Rs, Rs)], bounce)
        pltpu.sync_copy(bounce, out_hbm.at[pl.ds(c * R + s * Rs, Rs)])

    ones = jnp.ones((CH,), jnp.float32)
    return k(dst_p, zeros1d, ones).reshape(NC, R)


def _sc_aggregate(table, src_p, dst_p, zeros2d):
    R = zeros2d.shape[0]
    Rs = R // NS
    e_pad = src_p.shape[0]
    per_tile = e_pad // (NC * NS)
    n_chunks = per_tile // CH

    nj = 8
    half = Rs // nj

    @functools.partial(
        pl.kernel,
        out_type=jax.ShapeDtypeStruct((NC, R, LANES), jnp.float32),
        mesh=_sc_mesh(),
        compiler_params=_SC_PARAMS,
        scratch_types=[
            pltpu.VMEM((CH,), jnp.int32),
            pltpu.VMEM((CH,), jnp.int32),
            pltpu.VMEM((CH, LANES), jnp.float32),
            pltpu.VMEM((half, LANES), jnp.float32),
            pltpu.VMEM_SHARED((R, LANES), jnp.float32),
        ],
    )
    def k(table_hbm, src_hbm, dst_hbm, z_hbm, out_hbm, sidx, didx, rows, bounce, acc):
        c = lax.axis_index("c")
        s = lax.axis_index("s")
        for j in range(nj):
            pltpu.sync_copy(z_hbm.at[pl.ds(s * Rs + j * half, half)], bounce)
            pltpu.sync_copy(bounce, acc.at[pl.ds(s * Rs + j * half, half)])
        plsc.subcore_barrier()
        base = (c * NS + s) * per_tile

        @pl.loop(0, n_chunks)
        def _(i):
            off = base + i * CH
            pltpu.sync_copy(src_hbm.at[pl.ds(off, CH)], sidx)
            pltpu.sync_copy(dst_hbm.at[pl.ds(off, CH)], didx)
            pltpu.sync_copy(table_hbm.at[sidx], rows)
            pltpu.sync_copy(rows, acc.at[didx], add=True)

        plsc.subcore_barrier()
        for j in range(nj):
            pltpu.sync_copy(acc.at[pl.ds(s * Rs + j * half, half)], bounce)
            pltpu.sync_copy(
                bounce, out_hbm.at[c, pl.ds(s * Rs + j * half, half)]
            )

    return k(table, src_p, dst_p, zeros2d)


def _tc_scale(d0, d1, x, block_n):
    n = x.shape[0]

    def body(d0_ref, d1_ref, x_ref, xs_ref, di_ref):
        dv = lax.rsqrt(d0_ref[...] + d1_ref[...] + 1.0)
        di_ref[...] = dv
        xs_ref[...] = x_ref[...] * dv

    return pl.pallas_call(
        body,
        grid=(n // block_n,),
        in_specs=[
            pl.BlockSpec((block_n, 1), lambda i: (i, 0)),
            pl.BlockSpec((block_n, 1), lambda i: (i, 0)),
            pl.BlockSpec((block_n, LANES), lambda i: (i, 0)),
        ],
        out_specs=[
            pl.BlockSpec((block_n, LANES), lambda i: (i, 0)),
            pl.BlockSpec((block_n, 1), lambda i: (i, 0)),
        ],
        out_shape=[
            jax.ShapeDtypeStruct((n, LANES), jnp.float32),
            jax.ShapeDtypeStruct((n, 1), jnp.float32),
        ],
    )(d0, d1, x)


def _tc_layer(a0, a1, xs, di, W1, b1, W2p, block_n):
    n = xs.shape[0]
    d_hid = W1.shape[1]

    def body(a0_ref, a1_ref, xs_ref, di_ref, w1_ref, b1_ref, w2_ref, ts_ref):
        pre1 = di_ref[...] * (a0_ref[...] + a1_ref[...] + xs_ref[...])
        h = jnp.dot(pre1, w1_ref[...], preferred_element_type=jnp.float32)
        h = jnp.maximum(h + b1_ref[...], 0.0)
        ts_ref[...] = di_ref[...] * jnp.dot(
            h, w2_ref[...], preferred_element_type=jnp.float32
        )

    return pl.pallas_call(
        body,
        grid=(n // block_n,),
        in_specs=[
            pl.BlockSpec((block_n, LANES), lambda i: (i, 0)),
            pl.BlockSpec((block_n, LANES), lambda i: (i, 0)),
            pl.BlockSpec((block_n, LANES), lambda i: (i, 0)),
            pl.BlockSpec((block_n, 1), lambda i: (i, 0)),
            pl.BlockSpec((LANES, d_hid), lambda i: (0, 0)),
            pl.BlockSpec((1, d_hid), lambda i: (0, 0)),
            pl.BlockSpec((d_hid, LANES), lambda i: (0, 0)),
        ],
        out_specs=pl.BlockSpec((block_n, LANES), lambda i: (i, 0)),
        out_shape=jax.ShapeDtypeStruct((n, LANES), jnp.float32),
    )(a0, a1, xs, di, W1, b1, W2p)


def _tc_logsoftmax(a0, a1, ts, di, b2p, d_out, block_n):
    n = ts.shape[0]

    def body(a0_ref, a1_ref, ts_ref, di_ref, b2_ref, o_ref):
        v = di_ref[...] * (a0_ref[...] + a1_ref[...] + ts_ref[...]) + b2_ref[...]
        lane = lax.broadcasted_iota(jnp.int32, v.shape, 1)
        valid = lane < d_out
        m = jnp.max(jnp.where(valid, v, jnp.float32(-1e30)), axis=1, keepdims=True)
        e = jnp.where(valid, jnp.exp(v - m), 0.0)
        lse = m + jnp.log(jnp.sum(e, axis=1, keepdims=True))
        o_ref[...] = (v - lse)[:, :d_out]

    return pl.pallas_call(
        body,
        grid=(n // block_n,),
        in_specs=[
            pl.BlockSpec((block_n, LANES), lambda i: (i, 0)),
            pl.BlockSpec((block_n, LANES), lambda i: (i, 0)),
            pl.BlockSpec((block_n, LANES), lambda i: (i, 0)),
            pl.BlockSpec((block_n, 1), lambda i: (i, 0)),
            pl.BlockSpec((1, LANES), lambda i: (0, 0)),
        ],
        out_specs=pl.BlockSpec((block_n, d_out), lambda i: (i, 0)),
        out_shape=jax.ShapeDtypeStruct((n, d_out), jnp.float32),
    )(a0, a1, ts, di, b2p)


def kernel(x, edge_index, W1, b1, W2, b2):
    n, d_in = x.shape
    d_hid = W1.shape[1]
    d_out = W2.shape[1]
    e = edge_index.shape[1]

    tile_quant = NC * NS * CH
    e_pad = ((e + tile_quant - 1) // tile_quant) * tile_quant
    src_p = jnp.concatenate(
        [edge_index[0], jnp.zeros((e_pad - e,), jnp.int32)])
    dst_p = jnp.concatenate(
        [edge_index[1], jnp.full((e_pad - e,), n, jnp.int32)])

    R = ((n + 1 + NS * 8 - 1) // (NS * 8)) * (NS * 8)

    block_n = 2000

    degp = _sc_degree(dst_p, jnp.zeros((R,), jnp.float32))
    d0 = degp[0, :n].reshape(n, 1)
    d1 = degp[1, :n].reshape(n, 1)

    xs1, di = _tc_scale(d0, d1, x, block_n)

    zeros2d = jnp.zeros((R, LANES), jnp.float32)
    agg1 = _sc_aggregate(xs1, src_p, dst_p, zeros2d)

    W2p = jnp.concatenate(
        [W2, jnp.zeros((d_hid, LANES - d_out), jnp.float32)], axis=1)
    ts2 = _tc_layer(
        agg1[0, :n], agg1[1, :n], xs1, di, W1, b1.reshape(1, d_hid), W2p, block_n
    )

    agg2 = _sc_aggregate(ts2, src_p, dst_p, zeros2d)

    b2p = jnp.concatenate(
        [b2, jnp.zeros((LANES - d_out,), jnp.float32)]).reshape(1, LANES)
    return _tc_logsoftmax(agg2[0, :n], agg2[1, :n], ts2, di, b2p, d_out, block_n)


# speedup vs baseline: 39.1477x; 2.1360x over previous
import functools

import jax
import jax.numpy as jnp
from jax import lax
from jax.experimental import pallas as pl
from jax.experimental.pallas import tpu as pltpu
from jax.experimental.pallas import tpu_sc as plsc

NC = 2
NS = 16
LANES = 16
CH = 128


def _sc_mesh():
    return plsc.VectorSubcoreMesh(
        core_axis_name="c", subcore_axis_name="s", num_cores=NC, num_subcores=NS
    )


_SC_PARAMS = pltpu.CompilerParams(use_tc_tiling_on_sc=False)


NB = 8


def _sc_degree(edges_pk, zeros1d):
    R = zeros1d.shape[0]
    Rs = R // NS
    n_chunks_total = edges_pk.shape[0]
    per_tile = n_chunks_total // (NC * NS)

    @functools.partial(
        pl.kernel,
        out_type=jax.ShapeDtypeStruct((NC * R,), jnp.float32),
        mesh=_sc_mesh(),
        compiler_params=_SC_PARAMS,
        scratch_types=[
            pltpu.VMEM((NB, 2, CH), jnp.int32),
            pltpu.VMEM((CH,), jnp.float32),
            pltpu.VMEM((Rs,), jnp.float32),
            pltpu.VMEM_SHARED((R,), jnp.float32),
            pltpu.SemaphoreType.DMA((NB,)),
            pltpu.SemaphoreType.DMA((NB,)),
        ],
    )
    def k(e_hbm, z_hbm, ones_hbm, out_hbm, eb, ones_v, bounce, acc, isem, ssem):
        c = lax.axis_index("c")
        s = lax.axis_index("s")
        pltpu.sync_copy(z_hbm.at[pl.ds(s * Rs, Rs)], bounce)
        pltpu.sync_copy(bounce, acc.at[pl.ds(s * Rs, Rs)])
        pltpu.sync_copy(ones_hbm, ones_v)
        plsc.subcore_barrier()
        base = (c * NS + s) * per_tile

        @pl.loop(0, per_tile, step=NB)
        def _(i0):
            ids = [
                pltpu.async_copy(e_hbm.at[base + i0 + b], eb.at[b], isem.at[b])
                for b in range(NB)
            ]
            sds = []
            for b in range(NB):
                ids[b].wait()
                sds.append(
                    pltpu.async_copy(ones_v, acc.at[eb.at[b, 1]], ssem.at[b], add=True)
                )
            for b in range(NB):
                sds[b].wait()

        plsc.subcore_barrier()
        pltpu.sync_copy(acc.at[pl.ds(s * Rs, Rs)], bounce)
        pltpu.sync_copy(bounce, out_hbm.at[pl.ds(c * R + s * Rs, Rs)])

    ones = jnp.ones((CH,), jnp.float32)
    return k(edges_pk, zeros1d, ones).reshape(NC, R)


def _sc_aggregate(table, edges_pk, zeros2d):
    R = zeros2d.shape[0]
    Rs = R // NS
    n_chunks_total = edges_pk.shape[0]
    per_tile = n_chunks_total // (NC * NS)

    nj = 16
    bw = Rs // nj

    @functools.partial(
        pl.kernel,
        out_type=jax.ShapeDtypeStruct((NC, R, LANES), jnp.float32),
        mesh=_sc_mesh(),
        compiler_params=_SC_PARAMS,
        scratch_types=[
            pltpu.VMEM((NB, 2, CH), jnp.int32),
            pltpu.VMEM((NB, CH, LANES), jnp.float32),
            pltpu.VMEM((bw, LANES), jnp.float32),
            pltpu.VMEM_SHARED((R, LANES), jnp.float32),
            pltpu.SemaphoreType.DMA((NB,)),
            pltpu.SemaphoreType.DMA((NB,)),
            pltpu.SemaphoreType.DMA((NB,)),
        ],
    )
    def k(table_hbm, e_hbm, z_hbm, out_hbm, eb, rows, bounce, acc, isem, gsem, ssem):
        c = lax.axis_index("c")
        s = lax.axis_index("s")
        for j in range(nj):
            pltpu.sync_copy(z_hbm.at[pl.ds(s * Rs + j * bw, bw)], bounce)
            pltpu.sync_copy(bounce, acc.at[pl.ds(s * Rs + j * bw, bw)])
        plsc.subcore_barrier()
        base = (c * NS + s) * per_tile

        @pl.loop(0, per_tile, step=NB)
        def _(i0):
            ids = [
                pltpu.async_copy(e_hbm.at[base + i0 + b], eb.at[b], isem.at[b])
                for b in range(NB)
            ]
            gds = []
            for b in range(NB):
                ids[b].wait()
                gds.append(
                    pltpu.async_copy(table_hbm.at[eb.at[b, 0]], rows.at[b], gsem.at[b])
                )
            sds = []
            for b in range(NB):
                gds[b].wait()
                sds.append(
                    pltpu.async_copy(
                        rows.at[b], acc.at[eb.at[b, 1]], ssem.at[b], add=True
                    )
                )
            for b in range(NB):
                sds[b].wait()

        plsc.subcore_barrier()
        for j in range(nj):
            pltpu.sync_copy(acc.at[pl.ds(s * Rs + j * bw, bw)], bounce)
            pltpu.sync_copy(bounce, out_hbm.at[c, pl.ds(s * Rs + j * bw, bw)])

    return k(table, edges_pk, zeros2d)


def _tc_scale(d0, d1, x, block_n):
    n = x.shape[0]

    def body(d0_ref, d1_ref, x_ref, xs_ref, di_ref):
        dv = lax.rsqrt(d0_ref[...] + d1_ref[...] + 1.0)
        di_ref[...] = dv
        xs_ref[...] = x_ref[...] * dv

    return pl.pallas_call(
        body,
        grid=(n // block_n,),
        in_specs=[
            pl.BlockSpec((block_n, 1), lambda i: (i, 0)),
            pl.BlockSpec((block_n, 1), lambda i: (i, 0)),
            pl.BlockSpec((block_n, LANES), lambda i: (i, 0)),
        ],
        out_specs=[
            pl.BlockSpec((block_n, LANES), lambda i: (i, 0)),
            pl.BlockSpec((block_n, 1), lambda i: (i, 0)),
        ],
        out_shape=[
            jax.ShapeDtypeStruct((n, LANES), jnp.float32),
            jax.ShapeDtypeStruct((n, 1), jnp.float32),
        ],
    )(d0, d1, x)


def _tc_layer(a0, a1, xs, di, W1, b1, W2p, block_n):
    n = xs.shape[0]
    d_hid = W1.shape[1]

    def body(a0_ref, a1_ref, xs_ref, di_ref, w1_ref, b1_ref, w2_ref, ts_ref):
        pre1 = di_ref[...] * (a0_ref[...] + a1_ref[...] + xs_ref[...])
        h = jnp.dot(pre1, w1_ref[...], preferred_element_type=jnp.float32)
        h = jnp.maximum(h + b1_ref[...], 0.0)
        ts_ref[...] = di_ref[...] * jnp.dot(
            h, w2_ref[...], preferred_element_type=jnp.float32
        )

    return pl.pallas_call(
        body,
        grid=(n // block_n,),
        in_specs=[
            pl.BlockSpec((block_n, LANES), lambda i: (i, 0)),
            pl.BlockSpec((block_n, LANES), lambda i: (i, 0)),
            pl.BlockSpec((block_n, LANES), lambda i: (i, 0)),
            pl.BlockSpec((block_n, 1), lambda i: (i, 0)),
            pl.BlockSpec((LANES, d_hid), lambda i: (0, 0)),
            pl.BlockSpec((1, d_hid), lambda i: (0, 0)),
            pl.BlockSpec((d_hid, LANES), lambda i: (0, 0)),
        ],
        out_specs=pl.BlockSpec((block_n, LANES), lambda i: (i, 0)),
        out_shape=jax.ShapeDtypeStruct((n, LANES), jnp.float32),
    )(a0, a1, xs, di, W1, b1, W2p)


def _tc_logsoftmax(a0, a1, ts, di, b2p, d_out, block_n):
    n = ts.shape[0]

    def body(a0_ref, a1_ref, ts_ref, di_ref, b2_ref, o_ref):
        v = di_ref[...] * (a0_ref[...] + a1_ref[...] + ts_ref[...]) + b2_ref[...]
        lane = lax.broadcasted_iota(jnp.int32, v.shape, 1)
        valid = lane < d_out
        m = jnp.max(jnp.where(valid, v, jnp.float32(-1e30)), axis=1, keepdims=True)
        e = jnp.where(valid, jnp.exp(v - m), 0.0)
        lse = m + jnp.log(jnp.sum(e, axis=1, keepdims=True))
        o_ref[...] = (v - lse)[:, :d_out]

    return pl.pallas_call(
        body,
        grid=(n // block_n,),
        in_specs=[
            pl.BlockSpec((block_n, LANES), lambda i: (i, 0)),
            pl.BlockSpec((block_n, LANES), lambda i: (i, 0)),
            pl.BlockSpec((block_n, LANES), lambda i: (i, 0)),
            pl.BlockSpec((block_n, 1), lambda i: (i, 0)),
            pl.BlockSpec((1, LANES), lambda i: (0, 0)),
        ],
        out_specs=pl.BlockSpec((block_n, d_out), lambda i: (i, 0)),
        out_shape=jax.ShapeDtypeStruct((n, d_out), jnp.float32),
    )(a0, a1, ts, di, b2p)


def kernel(x, edge_index, W1, b1, W2, b2):
    n, d_in = x.shape
    d_hid = W1.shape[1]
    d_out = W2.shape[1]
    e = edge_index.shape[1]

    tile_quant = NC * NS * CH * NB
    e_pad = ((e + tile_quant - 1) // tile_quant) * tile_quant
    src_p = jnp.concatenate(
        [edge_index[0], jnp.zeros((e_pad - e,), jnp.int32)])
    dst_p = jnp.concatenate(
        [edge_index[1], jnp.full((e_pad - e,), n, jnp.int32)])
    edges_pk = jnp.stack(
        [src_p.reshape(-1, CH), dst_p.reshape(-1, CH)], axis=1)

    R = ((n + 1 + NS * 8 - 1) // (NS * 8)) * (NS * 8)

    block_n = 2000

    degp = _sc_degree(edges_pk, jnp.zeros((R,), jnp.float32))
    d0 = degp[0, :n].reshape(n, 1)
    d1 = degp[1, :n].reshape(n, 1)

    xs1, di = _tc_scale(d0, d1, x, block_n)

    zeros2d = jnp.zeros((R, LANES), jnp.float32)
    agg1 = _sc_aggregate(xs1, edges_pk, zeros2d)

    W2p = jnp.concatenate(
        [W2, jnp.zeros((d_hid, LANES - d_out), jnp.float32)], axis=1)
    ts2 = _tc_layer(
        agg1[0, :n], agg1[1, :n], xs1, di, W1, b1.reshape(1, d_hid), W2p, block_n
    )

    agg2 = _sc_aggregate(ts2, edges_pk, zeros2d)

    b2p = jnp.concatenate(
        [b2, jnp.zeros((LANES - d_out,), jnp.float32)]).reshape(1, LANES)
    return _tc_logsoftmax(agg2[0, :n], agg2[1, :n], ts2, di, b2p, d_out, block_n)


# speedup vs baseline: 66.6290x; 1.7020x over previous
import functools

import jax
import jax.numpy as jnp
from jax import lax
from jax.experimental import pallas as pl
from jax.experimental.pallas import tpu as pltpu
from jax.experimental.pallas import tpu_sc as plsc

NC = 2
NS = 16
LANES = 16
CH = 128
NB = 8


def _sc_mesh():
    return plsc.VectorSubcoreMesh(
        core_axis_name="c", subcore_axis_name="s", num_cores=NC, num_subcores=NS
    )


_SC_PARAMS = pltpu.CompilerParams(use_tc_tiling_on_sc=False)


def _sc_degree(dst2, zeros1d, R):
    Rs = R // NS
    n_chunks_total = dst2.shape[0]
    per_tile = n_chunks_total // (NC * NS)

    @functools.partial(
        pl.kernel,
        out_type=jax.ShapeDtypeStruct((NC * R,), jnp.float32),
        mesh=_sc_mesh(),
        compiler_params=_SC_PARAMS,
        scratch_types=[
            pltpu.VMEM((NB, CH), jnp.int32),
            pltpu.VMEM((CH,), jnp.float32),
            pltpu.VMEM((Rs,), jnp.float32),
            pltpu.VMEM_SHARED((R,), jnp.float32),
            pltpu.SemaphoreType.DMA,
            pltpu.SemaphoreType.DMA((NB,)),
        ],
    )
    def k(dst_hbm, z_hbm, ones_hbm, out_hbm, db, ones_v, bounce, acc, isem, ssem):
        c = lax.axis_index("c")
        s = lax.axis_index("s")
        pltpu.sync_copy(z_hbm, bounce)
        pltpu.sync_copy(bounce, acc.at[pl.ds(s * Rs, Rs)])
        pltpu.sync_copy(ones_hbm, ones_v)
        plsc.subcore_barrier()
        base = (c * NS + s) * per_tile

        @pl.loop(0, per_tile, step=NB)
        def _(i0):
            idma = pltpu.async_copy(dst_hbm.at[pl.ds(base + i0, NB)], db, isem)
            idma.wait()
            sds = [
                pltpu.async_copy(ones_v, acc.at[db.at[b]], ssem.at[b], add=True)
                for b in range(NB)
            ]
            for b in range(NB):
                sds[b].wait()

        plsc.subcore_barrier()
        pltpu.sync_copy(acc.at[pl.ds(s * Rs, Rs)], bounce)
        pltpu.sync_copy(bounce, out_hbm.at[pl.ds(c * R + s * Rs, Rs)])

    ones = jnp.ones((CH,), jnp.float32)
    return k(dst2, zeros1d, ones)


def _sc_aggregate(table, src2, dst2, zeros2d):
    R = table.shape[0]
    Rs = R // NS
    n_chunks_total = src2.shape[0]
    per_tile = n_chunks_total // (NC * NS)

    nj = 16
    bw = Rs // nj

    @functools.partial(
        pl.kernel,
        out_type=jax.ShapeDtypeStruct((NC, R, LANES), jnp.float32),
        mesh=_sc_mesh(),
        compiler_params=_SC_PARAMS,
        scratch_types=[
            pltpu.VMEM((NB, CH), jnp.int32),
            pltpu.VMEM((NB, CH), jnp.int32),
            pltpu.VMEM((NB, CH, LANES), jnp.float32),
            pltpu.VMEM((bw, LANES), jnp.float32),
            pltpu.VMEM_SHARED((R, LANES), jnp.float32),
            pltpu.SemaphoreType.DMA,
            pltpu.SemaphoreType.DMA((NB,)),
            pltpu.SemaphoreType.DMA((NB,)),
        ],
    )
    def k(table_hbm, src_hbm, dst_hbm, z_hbm, out_hbm,
          sb, db, rows, bounce, acc, isem, gsem, ssem):
        c = lax.axis_index("c")
        s = lax.axis_index("s")
        pltpu.sync_copy(z_hbm, bounce)
        for j in range(nj):
            pltpu.sync_copy(bounce, acc.at[pl.ds(s * Rs + j * bw, bw)])
        plsc.subcore_barrier()
        base = (c * NS + s) * per_tile

        @pl.loop(0, per_tile, step=NB)
        def _(i0):
            i1 = pltpu.async_copy(src_hbm.at[pl.ds(base + i0, NB)], sb, isem)
            i2 = pltpu.async_copy(dst_hbm.at[pl.ds(base + i0, NB)], db, isem)
            i1.wait()
            i2.wait()
            gds = [
                pltpu.async_copy(table_hbm.at[sb.at[b]], rows.at[b], gsem.at[b])
                for b in range(NB)
            ]
            sds = []
            for b in range(NB):
                gds[b].wait()
                sds.append(
                    pltpu.async_copy(
                        rows.at[b], acc.at[db.at[b]], ssem.at[b], add=True
                    )
                )
            for b in range(NB):
                sds[b].wait()

        plsc.subcore_barrier()
        for j in range(nj):
            pltpu.sync_copy(acc.at[pl.ds(s * Rs + j * bw, bw)], bounce)
            pltpu.sync_copy(bounce, out_hbm.at[c, pl.ds(s * Rs + j * bw, bw)])

    return k(table, src2, dst2, zeros2d)


def _tc_scale(deg_flat, x_view, R):
    dv_rows = R // CH
    vrows = R * LANES // CH
    degv = deg_flat.reshape(NC, dv_rows, CH)

    BQ = 16
    grid = dv_rows // BQ

    def body(d0_ref, d1_ref, x_ref, xs_ref, di_ref):
        dv = lax.rsqrt(d0_ref[0] + d1_ref[0] + 1.0)
        dve = jnp.broadcast_to(
            dv.reshape(BQ, LANES, 8, 1), (BQ, LANES, 8, LANES)
        ).reshape(BQ * LANES, CH)
        di_ref[...] = dve
        xs_ref[...] = x_ref[...] * dve

    return pl.pallas_call(
        body,
        grid=(grid,),
        in_specs=[
            pl.BlockSpec((1, BQ, CH), lambda i: (0, i, 0)),
            pl.BlockSpec((1, BQ, CH), lambda i: (1, i, 0)),
            pl.BlockSpec((BQ * LANES, CH), lambda i: (i, 0)),
        ],
        out_specs=[
            pl.BlockSpec((BQ * LANES, CH), lambda i: (i, 0)),
            pl.BlockSpec((BQ * LANES, CH), lambda i: (i, 0)),
        ],
        out_shape=[
            jax.ShapeDtypeStruct((vrows, CH), jnp.float32),
            jax.ShapeDtypeStruct((vrows, CH), jnp.float32),
        ],
    )(degv, degv, x_view)


def _tc_layer(agg1v, xs_view, di_view, W1big, b1big, W2big):
    vrows = xs_view.shape[0]
    BV = 784
    grid = vrows // BV

    def body(a0_ref, a1_ref, xs_ref, di_ref, w1_ref, b1_ref, w2_ref, ts_ref):
        di = di_ref[...]
        pre = di * (a0_ref[0] + a1_ref[0] + xs_ref[...])
        h = jnp.dot(pre, w1_ref[...], preferred_element_type=jnp.float32)
        h = jnp.maximum(h + b1_ref[...], 0.0)
        ts_ref[...] = di * jnp.dot(
            h, w2_ref[...], preferred_element_type=jnp.float32
        )

    return pl.pallas_call(
        body,
        grid=(grid,),
        in_specs=[
            pl.BlockSpec((1, BV, CH), lambda i: (0, i, 0)),
            pl.BlockSpec((1, BV, CH), lambda i: (1, i, 0)),
            pl.BlockSpec((BV, CH), lambda i: (i, 0)),
            pl.BlockSpec((BV, CH), lambda i: (i, 0)),
            pl.BlockSpec((CH, 2 * CH), lambda i: (0, 0)),
            pl.BlockSpec((1, 2 * CH), lambda i: (0, 0)),
            pl.BlockSpec((2 * CH, CH), lambda i: (0, 0)),
        ],
        out_specs=pl.BlockSpec((BV, CH), lambda i: (i, 0)),
        out_shape=jax.ShapeDtypeStruct((vrows, CH), jnp.float32),
    )(agg1v, agg1v, xs_view, di_view, W1big, b1big, W2big)


def _tc_head(agg2v, ts_view, di_view, b2big, n, d_out):
    BV = 112
    vrows = ts_view.shape[0]
    grid = vrows // BV

    def body(a0_ref, a1_ref, ts_ref, di_ref, b2_ref, o_ref):
        v = di_ref[...] * (a0_ref[0] + a1_ref[0] + ts_ref[...]) + b2_ref[...]
        vv = v.reshape(BV, 8, LANES)
        lane = lax.broadcasted_iota(jnp.int32, vv.shape, 2)
        valid = lane < d_out
        m = jnp.max(jnp.where(valid, vv, jnp.float32(-1e30)), axis=2,
                    keepdims=True)
        e = jnp.where(valid, jnp.exp(vv - m), 0.0)
        lse = m + jnp.log(jnp.sum(e, axis=2, keepdims=True))
        o_ref[...] = (vv - lse).reshape(BV * 8, LANES)[:, :d_out]

    return pl.pallas_call(
        body,
        grid=(grid,),
        in_specs=[
            pl.BlockSpec((1, BV, CH), lambda i: (0, i, 0)),
            pl.BlockSpec((1, BV, CH), lambda i: (1, i, 0)),
            pl.BlockSpec((BV, CH), lambda i: (i, 0)),
            pl.BlockSpec((BV, CH), lambda i: (i, 0)),
            pl.BlockSpec((1, CH), lambda i: (0, 0)),
        ],
        out_specs=pl.BlockSpec((BV * 8, d_out), lambda i: (i, 0)),
        out_shape=jax.ShapeDtypeStruct((n, d_out), jnp.float32),
    )(agg2v, agg2v, ts_view, di_view, b2big)


def kernel(x, edge_index, W1, b1, W2, b2):
    n, d_in = x.shape
    d_hid = W1.shape[1]
    d_out = W2.shape[1]
    e = edge_index.shape[1]

    R = 100352
    assert n <= R - 1 and R % 1024 == 0

    tile_quant = NC * NS * CH * NB
    e_pad = ((e + tile_quant - 1) // tile_quant) * tile_quant
    src2 = jnp.concatenate(
        [edge_index[0], jnp.zeros((e_pad - e,), jnp.int32)]).reshape(-1, CH)
    dst2 = jnp.concatenate(
        [edge_index[1], jnp.full((e_pad - e,), n, jnp.int32)]).reshape(-1, CH)

    deg_flat = _sc_degree(dst2, jnp.zeros((R // NS,), jnp.float32), R)

    x_view = jnp.pad(x, ((0, R - n), (0, 0))).reshape(R * LANES // CH, CH)
    xs1_view, di_view = _tc_scale(deg_flat, x_view, R)
    xs1 = xs1_view.reshape(R, LANES)

    zeros2d = jnp.zeros((R // NS // 16, LANES), jnp.float32)
    agg1 = _sc_aggregate(xs1, src2, dst2, zeros2d)
    agg1v = agg1.reshape(NC, R * LANES // CH, CH)

    W2p = jnp.concatenate(
        [W2, jnp.zeros((d_hid, LANES - d_out), jnp.float32)], axis=1)
    eye8 = jnp.eye(8, dtype=jnp.float32)
    W1big = jnp.kron(eye8, W1)
    W2big = jnp.kron(eye8, W2p)
    b1big = jnp.tile(b1, 8).reshape(1, 8 * d_hid)
    ts2_view = _tc_layer(agg1v, xs1_view, di_view, W1big, b1big, W2big)
    ts2 = ts2_view.reshape(R, LANES)

    agg2 = _sc_aggregate(ts2, src2, dst2, zeros2d)
    agg2v = agg2.reshape(NC, R * LANES // CH, CH)

    b2p = jnp.concatenate([b2, jnp.zeros((LANES - d_out,), jnp.float32)])
    b2big = jnp.tile(b2p, 8).reshape(1, CH)
    return _tc_head(agg2v, ts2_view, di_view, b2big, n, d_out)


# speedup vs baseline: 70.8111x; 1.0628x over previous
import functools

import jax
import jax.numpy as jnp
from jax import lax
from jax.experimental import pallas as pl
from jax.experimental.pallas import tpu as pltpu
from jax.experimental.pallas import tpu_sc as plsc

NC = 2
NS = 16
LANES = 16
CH = 128
NB = 8


def _sc_mesh():
    return plsc.VectorSubcoreMesh(
        core_axis_name="c", subcore_axis_name="s", num_cores=NC, num_subcores=NS
    )


_SC_PARAMS = pltpu.CompilerParams(use_tc_tiling_on_sc=False)


def _sc_degree(dst2, zeros1d, R):
    Rs = R // NS
    n_chunks_total = dst2.shape[0]
    per_tile = n_chunks_total // (NC * NS)

    @functools.partial(
        pl.kernel,
        out_type=jax.ShapeDtypeStruct((NC * R,), jnp.float32),
        mesh=_sc_mesh(),
        compiler_params=_SC_PARAMS,
        scratch_types=[
            pltpu.VMEM((NB, CH), jnp.int32),
            pltpu.VMEM((CH,), jnp.float32),
            pltpu.VMEM((Rs,), jnp.float32),
            pltpu.VMEM_SHARED((R,), jnp.float32),
            pltpu.SemaphoreType.DMA,
            pltpu.SemaphoreType.DMA((NB,)),
        ],
    )
    def k(dst_hbm, z_hbm, ones_hbm, out_hbm, db, ones_v, bounce, acc, isem, ssem):
        c = lax.axis_index("c")
        s = lax.axis_index("s")
        pltpu.sync_copy(z_hbm, bounce)
        pltpu.sync_copy(bounce, acc.at[pl.ds(s * Rs, Rs)])
        pltpu.sync_copy(ones_hbm, ones_v)
        plsc.subcore_barrier()
        base = (c * NS + s) * per_tile

        @pl.loop(0, per_tile, step=NB)
        def _(i0):
            idma = pltpu.async_copy(dst_hbm.at[pl.ds(base + i0, NB)], db, isem)
            idma.wait()
            sds = [
                pltpu.async_copy(ones_v, acc.at[db.at[b]], ssem.at[b], add=True)
                for b in range(NB)
            ]
            for b in range(NB):
                sds[b].wait()

        plsc.subcore_barrier()
        pltpu.sync_copy(acc.at[pl.ds(s * Rs, Rs)], bounce)
        pltpu.sync_copy(bounce, out_hbm.at[pl.ds(c * R + s * Rs, Rs)])

    ones = jnp.ones((CH,), jnp.float32)
    return k(dst2, zeros1d, ones)


def _sc_aggregate(table, src2, dst2, zeros2d):
    R = table.shape[0]
    Rs = R // NS
    n_chunks_total = src2.shape[0]
    per_tile = n_chunks_total // (NC * NS)

    nj = 16
    bw = Rs // nj

    @functools.partial(
        pl.kernel,
        out_type=jax.ShapeDtypeStruct((NC, R, LANES), jnp.float32),
        mesh=_sc_mesh(),
        compiler_params=_SC_PARAMS,
        scratch_types=[
            pltpu.VMEM((NB, CH), jnp.int32),
            pltpu.VMEM((NB, CH), jnp.int32),
            pltpu.VMEM((NB, CH, LANES), jnp.float32),
            pltpu.VMEM((bw, LANES), jnp.float32),
            pltpu.VMEM_SHARED((R, LANES), jnp.float32),
            pltpu.SemaphoreType.DMA,
            pltpu.SemaphoreType.DMA((NB,)),
            pltpu.SemaphoreType.DMA((NB,)),
        ],
    )
    def k(table_hbm, src_hbm, dst_hbm, z_hbm, out_hbm,
          sb, db, rows, bounce, acc, isem, gsem, ssem):
        c = lax.axis_index("c")
        s = lax.axis_index("s")
        pltpu.sync_copy(z_hbm, bounce)
        for j in range(nj):
            pltpu.sync_copy(bounce, acc.at[pl.ds(s * Rs + j * bw, bw)])
        plsc.subcore_barrier()
        base = (c * NS + s) * per_tile

        @pl.loop(0, per_tile, step=NB)
        def _(i0):
            i1 = pltpu.async_copy(src_hbm.at[pl.ds(base + i0, NB)], sb, isem)
            i2 = pltpu.async_copy(dst_hbm.at[pl.ds(base + i0, NB)], db, isem)
            i1.wait()
            i2.wait()
            gds = [
                pltpu.async_copy(table_hbm.at[sb.at[b]], rows.at[b], gsem.at[b])
                for b in range(NB)
            ]
            sds = []
            for b in range(NB):
                gds[b].wait()
                sds.append(
                    pltpu.async_copy(
                        rows.at[b], acc.at[db.at[b]], ssem.at[b], add=True
                    )
                )
            for b in range(NB):
                sds[b].wait()

        plsc.subcore_barrier()
        for j in range(nj):
            pltpu.sync_copy(acc.at[pl.ds(s * Rs + j * bw, bw)], bounce)
            pltpu.sync_copy(bounce, out_hbm.at[c, pl.ds(s * Rs + j * bw, bw)])

    return k(table, src2, dst2, zeros2d)


def _tc_scale(deg_flat, x_flat, R):
    dv_rows = R // CH
    vrows = R * LANES // CH
    degv = deg_flat.reshape(NC, dv_rows, CH)

    BQ = 16
    BF = BQ * LANES
    grid = dv_rows // BQ

    def body(d0_ref, d1_ref, x_ref, xs_ref, di_ref):
        dv = lax.rsqrt(d0_ref[0] + d1_ref[0] + 1.0)
        dve = jnp.broadcast_to(
            dv.reshape(BQ, LANES, 8, 1), (BQ, LANES, 8, LANES)
        ).reshape(BF, CH)
        di_ref[...] = dve
        xs_ref[...] = x_ref[...].reshape(BF, CH) * dve

    return pl.pallas_call(
        body,
        grid=(grid,),
        in_specs=[
            pl.BlockSpec((1, BQ, CH), lambda i: (0, i, 0)),
            pl.BlockSpec((1, BQ, CH), lambda i: (1, i, 0)),
            pl.BlockSpec((BF * CH,), lambda i: (i,)),
        ],
        out_specs=[
            pl.BlockSpec((BF, CH), lambda i: (i, 0)),
            pl.BlockSpec((BF, CH), lambda i: (i, 0)),
        ],
        out_shape=[
            jax.ShapeDtypeStruct((vrows, CH), jnp.float32),
            jax.ShapeDtypeStruct((vrows, CH), jnp.float32),
        ],
    )(degv, degv, x_flat)


def _tc_layer(agg1v, xs_view, di_view, W1big, b1big, W2big):
    vrows = xs_view.shape[0]
    BV = 784
    grid = vrows // BV

    def body(a0_ref, a1_ref, xs_ref, di_ref, w1_ref, b1_ref, w2_ref, ts_ref):
        di = di_ref[...]
        pre = di * (a0_ref[0] + a1_ref[0] + xs_ref[...])
        h = jnp.dot(pre, w1_ref[...], preferred_element_type=jnp.float32)
        h = jnp.maximum(h + b1_ref[...], 0.0)
        ts_ref[...] = di * jnp.dot(
            h, w2_ref[...], preferred_element_type=jnp.float32
        )

    return pl.pallas_call(
        body,
        grid=(grid,),
        in_specs=[
            pl.BlockSpec((1, BV, CH), lambda i: (0, i, 0)),
            pl.BlockSpec((1, BV, CH), lambda i: (1, i, 0)),
            pl.BlockSpec((BV, CH), lambda i: (i, 0)),
            pl.BlockSpec((BV, CH), lambda i: (i, 0)),
            pl.BlockSpec((CH, 2 * CH), lambda i: (0, 0)),
            pl.BlockSpec((1, 2 * CH), lambda i: (0, 0)),
            pl.BlockSpec((2 * CH, CH), lambda i: (0, 0)),
        ],
        out_specs=pl.BlockSpec((BV, CH), lambda i: (i, 0)),
        out_shape=jax.ShapeDtypeStruct((vrows, CH), jnp.float32),
    )(agg1v, agg1v, xs_view, di_view, W1big, b1big, W2big)


def _tc_head(agg2v, ts_view, di_view, b2big, n, d_out):
    BV = 448
    vrows = ts_view.shape[0]
    grid = vrows // BV

    def body(a0_ref, a1_ref, ts_ref, di_ref, b2_ref, o_ref):
        v = di_ref[...] * (a0_ref[0] + a1_ref[0] + ts_ref[...]) + b2_ref[...]
        lane = lax.broadcasted_iota(jnp.int32, v.shape, 1) % LANES
        r1 = jnp.roll(v, -1, axis=1)
        r2 = jnp.roll(v, -2, axis=1)
        mg = jnp.maximum(jnp.maximum(v, r1), r2)
        m = jnp.where(lane == 1, jnp.roll(mg, 1, axis=1), mg)
        m = jnp.where(lane == 2, jnp.roll(mg, 2, axis=1), m)
        e = jnp.where(lane < d_out, jnp.exp(v - m), 0.0)
        sg = e + jnp.roll(e, -1, axis=1) + jnp.roll(e, -2, axis=1)
        s = jnp.where(lane == 1, jnp.roll(sg, 1, axis=1), sg)
        s = jnp.where(lane == 2, jnp.roll(sg, 2, axis=1), s)
        res = v - m - jnp.log(s)
        o_ref[...] = jnp.concatenate(
            [res[:, k * LANES:k * LANES + d_out] for k in range(8)], axis=1
        )

    packed = pl.pallas_call(
        body,
        grid=(grid,),
        in_specs=[
            pl.BlockSpec((1, BV, CH), lambda i: (0, i, 0)),
            pl.BlockSpec((1, BV, CH), lambda i: (1, i, 0)),
            pl.BlockSpec((BV, CH), lambda i: (i, 0)),
            pl.BlockSpec((BV, CH), lambda i: (i, 0)),
            pl.BlockSpec((1, CH), lambda i: (0, 0)),
        ],
        out_specs=pl.BlockSpec((BV, 8 * d_out), lambda i: (i, 0)),
        out_shape=jax.ShapeDtypeStruct((n // 8, 8 * d_out), jnp.float32),
    )(agg2v, agg2v, ts_view, di_view, b2big)
    return packed.reshape(n, d_out)


def kernel(x, edge_index, W1, b1, W2, b2):
    n, d_in = x.shape
    d_hid = W1.shape[1]
    d_out = W2.shape[1]
    e = edge_index.shape[1]

    R = 100352
    assert n <= R - 1 and R % 1024 == 0

    tile_quant = NC * NS * CH * NB
    e_pad = ((e + tile_quant - 1) // tile_quant) * tile_quant
    dst2 = jnp.concatenate(
        [edge_index[1], jnp.full((e_pad - e,), n, jnp.int32)]).reshape(-1, CH)

    deg_flat = _sc_degree(dst2, jnp.zeros((R // NS,), jnp.float32), R)
    src2 = jnp.concatenate(
        [edge_index[0], jnp.zeros((e_pad - e,), jnp.int32)]).reshape(-1, CH)

    x_flat = x.reshape(-1)
    xs1_view, di_view = _tc_scale(deg_flat, x_flat, R)
    xs1 = xs1_view.reshape(R, LANES)

    zeros2d = jnp.zeros((R // NS // 16, LANES), jnp.float32)
    agg1 = _sc_aggregate(xs1, src2, dst2, zeros2d)
    agg1v = agg1.reshape(NC, R * LANES // CH, CH)

    W2p = jnp.concatenate(
        [W2, jnp.zeros((d_hid, LANES - d_out), jnp.float32)], axis=1)
    eye8 = jnp.eye(8, dtype=jnp.float32)
    W1big = jnp.kron(eye8, W1)
    W2big = jnp.kron(eye8, W2p)
    b1big = jnp.tile(b1, 8).reshape(1, 8 * d_hid)
    ts2_view = _tc_layer(agg1v, xs1_view, di_view, W1big, b1big, W2big)
    ts2 = ts2_view.reshape(R, LANES)

    agg2 = _sc_aggregate(ts2, src2, dst2, zeros2d)
    agg2v = agg2.reshape(NC, R * LANES // CH, CH)

    b2p = jnp.concatenate([b2, jnp.zeros((LANES - d_out,), jnp.float32)])
    b2big = jnp.tile(b2p, 8).reshape(1, CH)
    return _tc_head(agg2v, ts2_view, di_view, b2big, n, d_out)


# speedup vs baseline: 71.4087x; 1.0084x over previous
import functools

import jax
import jax.numpy as jnp
from jax import lax
from jax.experimental import pallas as pl
from jax.experimental.pallas import tpu as pltpu
from jax.experimental.pallas import tpu_sc as plsc

NC = 2
NS = 16
LANES = 16
CH = 128
NB = 8


def _sc_mesh():
    return plsc.VectorSubcoreMesh(
        core_axis_name="c", subcore_axis_name="s", num_cores=NC, num_subcores=NS
    )


_SC_PARAMS = pltpu.CompilerParams(use_tc_tiling_on_sc=False)


def _sc_degree(e2, tails, zeros1d, R):
    Rs = R // NS
    per_tile = tails.shape[1]
    last = NC * NS - 1

    @functools.partial(
        pl.kernel,
        out_type=jax.ShapeDtypeStruct((NC * R,), jnp.float32),
        mesh=_sc_mesh(),
        compiler_params=_SC_PARAMS,
        scratch_types=[
            pltpu.VMEM((NB, CH), jnp.int32),
            pltpu.VMEM((CH,), jnp.float32),
            pltpu.VMEM((Rs,), jnp.float32),
            pltpu.VMEM_SHARED((R,), jnp.float32),
            pltpu.SemaphoreType.DMA,
            pltpu.SemaphoreType.DMA((NB,)),
        ],
    )
    def k(e_hbm, t_hbm, z_hbm, ones_hbm, out_hbm, db, ones_v, bounce, acc,
          isem, ssem):
        c = lax.axis_index("c")
        s = lax.axis_index("s")
        pltpu.sync_copy(z_hbm, bounce)
        pltpu.sync_copy(bounce, acc.at[pl.ds(s * Rs, Rs)])
        pltpu.sync_copy(ones_hbm, ones_v)
        plsc.subcore_barrier()
        tid = c * NS + s
        base = tid * per_tile

        @pl.loop(0, per_tile, step=NB)
        def _(i0):
            @pl.when(tid == last)
            def _():
                pltpu.sync_copy(t_hbm.at[1, pl.ds(i0, NB)], db)

            @pl.when(tid != last)
            def _():
                pltpu.sync_copy(e_hbm.at[1, pl.ds(base + i0, NB)], db)

            sds = [
                pltpu.async_copy(ones_v, acc.at[db.at[b]], ssem.at[b], add=True)
                for b in range(NB)
            ]
            for b in range(NB):
                sds[b].wait()

        plsc.subcore_barrier()
        pltpu.sync_copy(acc.at[pl.ds(s * Rs, Rs)], bounce)
        pltpu.sync_copy(bounce, out_hbm.at[pl.ds(c * R + s * Rs, Rs)])

    ones = jnp.ones((CH,), jnp.float32)
    return k(e2, tails, zeros1d, ones)


def _sc_aggregate(table, e2, tails, zeros2d):
    R = table.shape[0]
    Rs = R // NS
    per_tile = tails.shape[1]
    last = NC * NS - 1

    nj = 16
    bw = Rs // nj

    @functools.partial(
        pl.kernel,
        out_type=jax.ShapeDtypeStruct((NC, R, LANES), jnp.float32),
        mesh=_sc_mesh(),
        compiler_params=_SC_PARAMS,
        scratch_types=[
            pltpu.VMEM((NB, CH), jnp.int32),
            pltpu.VMEM((NB, CH), jnp.int32),
            pltpu.VMEM((NB, CH, LANES), jnp.float32),
            pltpu.VMEM((bw, LANES), jnp.float32),
            pltpu.VMEM_SHARED((R, LANES), jnp.float32),
            pltpu.SemaphoreType.DMA,
            pltpu.SemaphoreType.DMA((NB,)),
            pltpu.SemaphoreType.DMA((NB,)),
        ],
    )
    def k(table_hbm, e_hbm, t_hbm, z_hbm, out_hbm,
          sb, db, rows, bounce, acc, isem, gsem, ssem):
        c = lax.axis_index("c")
        s = lax.axis_index("s")
        pltpu.sync_copy(z_hbm, bounce)
        for j in range(nj):
            pltpu.sync_copy(bounce, acc.at[pl.ds(s * Rs + j * bw, bw)])
        plsc.subcore_barrier()
        tid = c * NS + s
        base = tid * per_tile

        @pl.loop(0, per_tile, step=NB)
        def _(i0):
            @pl.when(tid == last)
            def _():
                i1 = pltpu.async_copy(t_hbm.at[0, pl.ds(i0, NB)], sb, isem)
                i2 = pltpu.async_copy(t_hbm.at[1, pl.ds(i0, NB)], db, isem)
                i1.wait()
                i2.wait()

            @pl.when(tid != last)
            def _():
                i1 = pltpu.async_copy(
                    e_hbm.at[0, pl.ds(base + i0, NB)], sb, isem)
                i2 = pltpu.async_copy(
                    e_hbm.at[1, pl.ds(base + i0, NB)], db, isem)
                i1.wait()
                i2.wait()

            gds = [
                pltpu.async_copy(table_hbm.at[sb.at[b]], rows.at[b], gsem.at[b])
                for b in range(NB)
            ]
            sds = []
            for b in range(NB):
                gds[b].wait()
                sds.append(
                    pltpu.async_copy(
                        rows.at[b], acc.at[db.at[b]], ssem.at[b], add=True
                    )
                )
            for b in range(NB):
                sds[b].wait()

        plsc.subcore_barrier()
        for j in range(nj):
            pltpu.sync_copy(acc.at[pl.ds(s * Rs + j * bw, bw)], bounce)
            pltpu.sync_copy(bounce, out_hbm.at[c, pl.ds(s * Rs + j * bw, bw)])

    return k(table, e2, tails, zeros2d)


def _tc_scale(deg_flat, x_flat, R):
    dv_rows = R // CH
    vrows = R * LANES // CH
    degv = deg_flat.reshape(NC, dv_rows, CH)

    BQ = 16
    BF = BQ * LANES
    grid = dv_rows // BQ

    def body(d0_ref, d1_ref, x_ref, xs_ref, di_ref):
        dv = lax.rsqrt(d0_ref[0] + d1_ref[0] + 1.0)
        dve = jnp.broadcast_to(
            dv.reshape(BQ, LANES, 8, 1), (BQ, LANES, 8, LANES)
        ).reshape(BF, CH)
        di_ref[...] = dve
        xs_ref[...] = x_ref[...].reshape(BF, CH) * dve

    return pl.pallas_call(
        body,
        grid=(grid,),
        in_specs=[
            pl.BlockSpec((1, BQ, CH), lambda i: (0, i, 0)),
            pl.BlockSpec((1, BQ, CH), lambda i: (1, i, 0)),
            pl.BlockSpec((BF * CH,), lambda i: (i,)),
        ],
        out_specs=[
            pl.BlockSpec((BF, CH), lambda i: (i, 0)),
            pl.BlockSpec((BF, CH), lambda i: (i, 0)),
        ],
        out_shape=[
            jax.ShapeDtypeStruct((vrows, CH), jnp.float32),
            jax.ShapeDtypeStruct((vrows, CH), jnp.float32),
        ],
    )(degv, degv, x_flat)


def _tc_layer(agg1v, xs_view, di_view, W1big, b1big, W2big):
    vrows = xs_view.shape[0]
    BV = 784
    grid = vrows // BV

    def body(a0_ref, a1_ref, xs_ref, di_ref, w1_ref, b1_ref, w2_ref, ts_ref):
        di = di_ref[...]
        pre = di * (a0_ref[0] + a1_ref[0] + xs_ref[...])
        h = jnp.dot(pre, w1_ref[...], preferred_element_type=jnp.float32)
        h = jnp.maximum(h + b1_ref[...], 0.0)
        ts_ref[...] = di * jnp.dot(
            h, w2_ref[...], preferred_element_type=jnp.float32
        )

    return pl.pallas_call(
        body,
        grid=(grid,),
        in_specs=[
            pl.BlockSpec((1, BV, CH), lambda i: (0, i, 0)),
            pl.BlockSpec((1, BV, CH), lambda i: (1, i, 0)),
            pl.BlockSpec((BV, CH), lambda i: (i, 0)),
            pl.BlockSpec((BV, CH), lambda i: (i, 0)),
            pl.BlockSpec((CH, 2 * CH), lambda i: (0, 0)),
            pl.BlockSpec((1, 2 * CH), lambda i: (0, 0)),
            pl.BlockSpec((2 * CH, CH), lambda i: (0, 0)),
        ],
        out_specs=pl.BlockSpec((BV, CH), lambda i: (i, 0)),
        out_shape=jax.ShapeDtypeStruct((vrows, CH), jnp.float32),
    )(agg1v, agg1v, xs_view, di_view, W1big, b1big, W2big)


def _tc_head(agg2v, ts_view, di_view, b2big, n, d_out):
    BV = 448
    vrows = ts_view.shape[0]
    grid = vrows // BV

    def body(a0_ref, a1_ref, ts_ref, di_ref, b2_ref, o_ref):
        v = di_ref[...] * (a0_ref[0] + a1_ref[0] + ts_ref[...]) + b2_ref[...]
        lane = lax.broadcasted_iota(jnp.int32, v.shape, 1) % LANES
        r1 = jnp.roll(v, -1, axis=1)
        r2 = jnp.roll(v, -2, axis=1)
        mg = jnp.maximum(jnp.maximum(v, r1), r2)
        m = jnp.where(lane == 1, jnp.roll(mg, 1, axis=1), mg)
        m = jnp.where(lane == 2, jnp.roll(mg, 2, axis=1), m)
        e = jnp.where(lane < d_out, jnp.exp(v - m), 0.0)
        sg = e + jnp.roll(e, -1, axis=1) + jnp.roll(e, -2, axis=1)
        s = jnp.where(lane == 1, jnp.roll(sg, 1, axis=1), sg)
        s = jnp.where(lane == 2, jnp.roll(sg, 2, axis=1), s)
        res = v - m - jnp.log(s)
        o_ref[...] = jnp.concatenate(
            [res[:, k * LANES:k * LANES + d_out] for k in range(8)], axis=1
        )

    packed = pl.pallas_call(
        body,
        grid=(grid,),
        in_specs=[
            pl.BlockSpec((1, BV, CH), lambda i: (0, i, 0)),
            pl.BlockSpec((1, BV, CH), lambda i: (1, i, 0)),
            pl.BlockSpec((BV, CH), lambda i: (i, 0)),
            pl.BlockSpec((BV, CH), lambda i: (i, 0)),
            pl.BlockSpec((1, CH), lambda i: (0, 0)),
        ],
        out_specs=pl.BlockSpec((BV, 8 * d_out), lambda i: (i, 0)),
        out_shape=jax.ShapeDtypeStruct((n // 8, 8 * d_out), jnp.float32),
    )(agg2v, agg2v, ts_view, di_view, b2big)
    return packed.reshape(n, d_out)


def kernel(x, edge_index, W1, b1, W2, b2):
    n, d_in = x.shape
    d_hid = W1.shape[1]
    d_out = W2.shape[1]
    e = edge_index.shape[1]

    R = 100352
    assert n <= R - 1 and R % 1024 == 0

    assert e % CH == 0
    n_chunks = e // CH
    PT = ((n_chunks + NC * NS * NB - 1) // (NC * NS * NB)) * NB
    last_base = (NC * NS - 1) * PT
    e2 = edge_index.reshape(2, n_chunks, CH)
    pad_chunks = NC * NS * PT - n_chunks
    tails = jnp.concatenate([
        e2[:, last_base:, :],
        jnp.stack([jnp.zeros((pad_chunks, CH), jnp.int32),
                   jnp.full((pad_chunks, CH), n, jnp.int32)]),
    ], axis=1)

    deg_flat = _sc_degree(e2, tails, jnp.zeros((R // NS,), jnp.float32), R)

    x_flat = x.reshape(-1)
    xs1_view, di_view = _tc_scale(deg_flat, x_flat, R)
    xs1 = xs1_view.reshape(R, LANES)

    zeros2d = jnp.zeros((R // NS // 16, LANES), jnp.float32)
    agg1 = _sc_aggregate(xs1, e2, tails, zeros2d)
    agg1v = agg1.reshape(NC, R * LANES // CH, CH)

    W2p = jnp.concatenate(
        [W2, jnp.zeros((d_hid, LANES - d_out), jnp.float32)], axis=1)
    eye8 = jnp.eye(8, dtype=jnp.float32)
    W1big = jnp.kron(eye8, W1)
    W2big = jnp.kron(eye8, W2p)
    b1big = jnp.tile(b1, 8).reshape(1, 8 * d_hid)
    ts2_view = _tc_layer(agg1v, xs1_view, di_view, W1big, b1big, W2big)
    ts2 = ts2_view.reshape(R, LANES)

    agg2 = _sc_aggregate(ts2, e2, tails, zeros2d)
    agg2v = agg2.reshape(NC, R * LANES // CH, CH)

    b2p = jnp.concatenate([b2, jnp.zeros((LANES - d_out,), jnp.float32)])
    b2big = jnp.tile(b2p, 8).reshape(1, CH)
    return _tc_head(agg2v, ts2_view, di_view, b2big, n, d_out)
